# Initial kernel scaffold; baseline (speedup 1.0000x reference)
import jax
import jax.numpy as jnp
from jax.experimental import pallas as pl


def kernel(keypoint, pc, sn):
    raise NotImplementedError("write your pallas kernel here")


# speedup vs baseline: 1.8591x; 1.8591x over previous
import jax
import jax.numpy as jnp
from jax.experimental import pallas as pl
from jax.experimental.pallas import tpu as pltpu


def _loss_kernel(kp_ref, pc_ref, sn_ref, out_ref):
    kp = kp_ref[0]
    pc = pc_ref[0]
    sn = sn_ref[0]

    d0 = kp[0][:, None] - pc[0][None, :]
    d1 = kp[1][:, None] - pc[1][None, :]
    d2 = kp[2][:, None] - pc[2][None, :]
    dist = jnp.sqrt(d0 * d0 + d1 * d1 + d2 * d2)
    mind = jnp.min(dist, axis=1, keepdims=True)

    iota = jax.lax.broadcasted_iota(jnp.int32, dist.shape, 1)
    idx = jnp.min(
        jnp.where(dist == mind, iota, jnp.int32(8192)), axis=1, keepdims=True
    )

    num_all = sn[0][None, :] * d0 + sn[1][None, :] * d1 + sn[2][None, :] * d2
    num = jnp.sum(jnp.where(iota == idx, num_all, 0.0), axis=1, keepdims=True)

    dot = num / (mind + 1e-7)
    out_ref[0] = (dot * dot).reshape(1, 128)


def kernel(keypoint, pc, sn):
    B, _, M = keypoint.shape
    out = pl.pallas_call(
        _loss_kernel,
        grid=(B,),
        in_specs=[
            pl.BlockSpec((1, 3, M), lambda b: (b, 0, 0)),
            pl.BlockSpec((1, 3, pc.shape[2]), lambda b: (b, 0, 0)),
            pl.BlockSpec((1, 3, sn.shape[2]), lambda b: (b, 0, 0)),
        ],
        out_specs=pl.BlockSpec((1, 1, M), lambda b: (b, 0, 0)),
        out_shape=jax.ShapeDtypeStruct((B, 1, M), jnp.float32),
    )(keypoint, pc, sn)
    return out.reshape(B, M, 1, 1)


# speedup vs baseline: 2.5620x; 1.3781x over previous
import jax
import jax.numpy as jnp
from jax.experimental import pallas as pl
from jax.experimental.pallas import tpu as pltpu


def _loss_kernel(kp_ref, pc_ref, sn_ref, out_ref):
    kp = kp_ref[0]
    pc = pc_ref[0]
    sn = sn_ref[0]

    d0 = kp[0][:, None] - pc[0][None, :]
    d1 = kp[1][:, None] - pc[1][None, :]
    d2 = kp[2][:, None] - pc[2][None, :]
    dist2 = d0 * d0 + d1 * d1 + d2 * d2
    mind2 = jnp.min(dist2, axis=1, keepdims=True)

    iota = jax.lax.broadcasted_iota(jnp.int32, dist2.shape, 1)
    idx = jnp.min(
        jnp.where(dist2 == mind2, iota, jnp.int32(8192)), axis=1, keepdims=True
    )

    onehot = (iota == idx).astype(jnp.float32)
    pc_sel = jax.lax.dot_general(
        onehot, pc, (((1,), (1,)), ((), ())), preferred_element_type=jnp.float32
    )
    sn_sel = jax.lax.dot_general(
        onehot, sn, (((1,), (1,)), ((), ())), preferred_element_type=jnp.float32
    )

    diff = kp.T - pc_sel
    num = jnp.sum(sn_sel * diff, axis=1, keepdims=True)
    dot = num / (jnp.sqrt(mind2) + 1e-7)
    out_ref[0] = (dot * dot).reshape(1, 128)


def kernel(keypoint, pc, sn):
    B, _, M = keypoint.shape
    out = pl.pallas_call(
        _loss_kernel,
        grid=(B,),
        in_specs=[
            pl.BlockSpec((1, 3, M), lambda b: (b, 0, 0)),
            pl.BlockSpec((1, 3, pc.shape[2]), lambda b: (b, 0, 0)),
            pl.BlockSpec((1, 3, sn.shape[2]), lambda b: (b, 0, 0)),
        ],
        out_specs=pl.BlockSpec((1, 1, M), lambda b: (b, 0, 0)),
        out_shape=jax.ShapeDtypeStruct((B, 1, M), jnp.float32),
    )(keypoint, pc, sn)
    return out.reshape(B, M, 1, 1)
